# baseline (speedup 1.0000x reference)
import functools

import jax
import jax.numpy as jnp
from jax import lax
from jax.experimental import pallas as pl
from jax.experimental.pallas import tpu as pltpu

E = 768
POOL = 100
K = 5
PLEN = 5
H = 12
HD = 64
L2 = 24
B = 4
SEQ = 2048
SEQ_CHUNK = 256
NCH = SEQ // SEQ_CHUNK
ROW = 320
RPE = (PLEN * H * HD) // ROW
RPB = 2 * K * RPE
NEG = -3.0e38


def _phase1_body(x_ref, skey_ref, mkey_ref,
                 ssim_ref, msim_ref, sidx_ref, midx_ref, rsum_ref):
    mean = jnp.sum(x_ref[...], axis=1) * (1.0 / SEQ)
    sq = jnp.sum(mean * mean, axis=-1, keepdims=True)
    xn = mean * lax.rsqrt(jnp.maximum(sq, 1e-12))

    kcol = lax.broadcasted_iota(jnp.int32, (B, K), 1)

    def pool_topk(key_ref, sim_ref, idx_ref):
        kv = key_ref[...]
        inv = lax.rsqrt(jnp.maximum(jnp.sum(kv * kv, axis=-1,
                                            keepdims=True), 1e-12))
        kn = kv * inv
        sim = lax.dot_general(xn.astype(jnp.bfloat16),
                              kn.astype(jnp.bfloat16),
                              (((1,), (1,)), ((), ())),
                              preferred_element_type=jnp.float32)
        sim_ref[...] = sim
        iota = lax.broadcasted_iota(jnp.int32, (B, POOL), 1)
        cur = sim
        idxs = jnp.zeros((B, K), jnp.int32)
        tot = jnp.float32(0.0)
        for t in range(K):
            mx = jnp.max(cur, axis=1, keepdims=True)
            pos = jnp.min(jnp.where(cur == mx, iota, POOL),
                          axis=1, keepdims=True)
            tot = tot + jnp.sum(mx)
            idxs = jnp.where(kcol == t, pos, idxs)
            cur = jnp.where(iota == pos, NEG, cur)
        idx_ref[...] = idxs
        return tot / B

    s_rs = pool_topk(skey_ref, ssim_ref, sidx_ref)
    m_rs = pool_topk(mkey_ref, msim_ref, midx_ref)
    two = lax.broadcasted_iota(jnp.int32, (1, 2), 1)
    rsum_ref[...] = jnp.where(two == 0, s_rs, m_rs)


def _phase1(x_embed, s_prompt_key, m_prompt_key):
    out_shapes = (
        jax.ShapeDtypeStruct((B, POOL), jnp.float32),
        jax.ShapeDtypeStruct((B, POOL), jnp.float32),
        jax.ShapeDtypeStruct((B, K), jnp.int32),
        jax.ShapeDtypeStruct((B, K), jnp.int32),
        jax.ShapeDtypeStruct((1, 2), jnp.float32),
    )
    full = lambda shape: pl.BlockSpec(shape, lambda: (0,) * len(shape))
    return pl.pallas_call(
        _phase1_body,
        in_specs=[
            full((B, SEQ, E)),
            full((POOL, E)),
            full((POOL, E)),
        ],
        out_specs=(
            full((B, POOL)), full((B, POOL)),
            full((B, K)), full((B, K)), full((1, 2)),
        ),
        out_shape=out_shapes,
    )(x_embed, s_prompt_key, m_prompt_key)


def _gather_body(sidx_ref, midx_ref, *refs):
    srefs = refs[0:K]
    mrefs = refs[K:2 * K]
    out_ref = refs[2 * K]
    for g in range(RPE):
        q0 = 5 * g
        k0, j0 = divmod(q0, RPE)
        for off, prefs in ((0, srefs), (5, mrefs)):
            if j0 + 5 <= RPE:
                piece = prefs[k0][0, 0, j0:j0 + 5, :]
            else:
                piece = jnp.concatenate(
                    [prefs[k0][0, 0, j0:RPE, :],
                     prefs[k0 + 1][0, 0, 0:j0 + 5 - RPE, :]], axis=0)
            out_ref[0, 0, 10 * g + off:10 * g + off + 5, :] = piece


def _phase2(s_prompt, m_prompt, s_idx, m_idx):
    s_p = s_prompt.reshape(L2, POOL, RPE, ROW)
    m_p = m_prompt.reshape(L2, POOL, RPE, ROW)

    def mk_spec(k, which):
        if which == 0:
            return pl.BlockSpec(
                (1, 1, RPE, ROW),
                lambda l, b, sidx, midx, kk=k: (l, sidx[b, kk], 0, 0))
        return pl.BlockSpec(
            (1, 1, RPE, ROW),
            lambda l, b, sidx, midx, kk=k: (l, midx[b, kk], 0, 0))

    grid_spec = pltpu.PrefetchScalarGridSpec(
        num_scalar_prefetch=2,
        grid=(L2, B),
        in_specs=[mk_spec(k, 0) for k in range(K)]
        + [mk_spec(k, 1) for k in range(K)],
        out_specs=pl.BlockSpec((1, 1, RPB, ROW),
                               lambda l, b, sidx, midx: (l, b, 0, 0)),
    )
    out = pl.pallas_call(
        _gather_body,
        grid_spec=grid_spec,
        out_shape=jax.ShapeDtypeStruct((L2, B, RPB, ROW), jnp.float32),
    )(s_idx, m_idx, *([s_p] * K), *([m_p] * K))
    return out.reshape(L2, B, H, 2 * K * PLEN, HD)


def kernel(x_embed, s_prompt, m_prompt, s_prompt_key, m_prompt_key):
    s_sim, m_sim, s_idx, m_idx, rsum = _phase1(
        x_embed, s_prompt_key, m_prompt_key)
    batched_prompt = _phase2(s_prompt, m_prompt, s_idx, m_idx)
    s_reduce = rsum[0, 0].reshape(())
    m_reduce = rsum[0, 1].reshape(())
    return (batched_prompt, s_sim, m_sim, s_reduce, m_reduce, s_idx, m_idx)
